# baseline (speedup 1.0000x reference)
import functools

import jax
import jax.numpy as jnp
from jax import lax
from jax.experimental import pallas as pl
from jax.experimental.pallas import tpu as pltpu
from jax.experimental.pallas import tpu_sc as plsc

N_SC = 2
N_TILES = 16
NW = N_SC * N_TILES


def _relu_body(x_ref, o_ref):
    o_ref[...] = jnp.maximum(x_ref[...], 0.0)


def _mlp_body(x_ref, p_ref, w1t_ref, b1_ref, g_ref, bt_ref, w2t_ref,
              b2_ref, eps_ref, o_ref):
    n = x_ref.shape[0]
    h = (1.0 + eps_ref[...]) * x_ref[...] + p_ref[0, :n] + p_ref[1, :n]
    l1 = jnp.dot(h, w1t_ref[...], preferred_element_type=jnp.float32)
    l1 = l1 + b1_ref[...]
    mean = jnp.mean(l1, axis=0, keepdims=True)
    cen = l1 - mean
    var = jnp.mean(cen * cen, axis=0, keepdims=True)
    hn = cen * lax.rsqrt(var + 1e-5) * g_ref[...] + bt_ref[...]
    hn = jnp.maximum(hn, 0.0)
    o_ref[...] = (jnp.dot(hn, w2t_ref[...], preferred_element_type=jnp.float32)
                  + b2_ref[...])


def _make_sc_scatter(N, D, nchunk, C, S, K):
    rows_per_tile = N // N_TILES
    nsec = nchunk // S
    assert S % K == 0
    mesh = plsc.VectorSubcoreMesh(
        core_axis_name="c", subcore_axis_name="s",
        num_cores=N_SC, num_subcores=N_TILES)

    @functools.partial(
        pl.kernel,
        out_type=jax.ShapeDtypeStruct((N_SC, N, D), jnp.float32),
        mesh=mesh,
        scratch_types=[
            pltpu.VMEM((2, S, C), jnp.int32),
            pltpu.VMEM((2, S, C), jnp.int32),
            pltpu.VMEM((K, C, D), jnp.float32),
            pltpu.VMEM_SHARED((N, D), jnp.float32),
            [pltpu.SemaphoreType.DMA] * K,
            pltpu.SemaphoreType.DMA,
            pltpu.SemaphoreType.DMA,
        ],
    )
    def sc_scatter(r_hbm, src_hbm, dst_hbm, z_hbm, out_hbm,
                   src_v, dst_v, rows_v, acc_sh, gsems, semis, semid):
        cid = lax.axis_index("c")
        sid = lax.axis_index("s")
        wid = sid * N_SC + cid
        r0 = sid * rows_per_tile

        with jax.named_scope("gin_zinit"):
            pltpu.sync_copy(z_hbm, acc_sh.at[pl.ds(r0, rows_per_tile)])

            pltpu.sync_copy(src_hbm.at[wid, pl.ds(0, S)], src_v.at[0])
            pltpu.sync_copy(dst_hbm.at[wid, pl.ds(0, S)], dst_v.at[0])
            pltpu.async_copy(src_hbm.at[wid, pl.ds(S, S)], src_v.at[1],
                             semis)
            pltpu.async_copy(dst_hbm.at[wid, pl.ds(S, S)], dst_v.at[1],
                             semid)
            plsc.subcore_barrier()

        for pb in range(K):
            pltpu.async_copy(r_hbm.at[src_v.at[0, pb]], rows_v.at[pb],
                             gsems[pb])

        def idx_wait(s, buf):
            pltpu.make_async_copy(
                src_hbm.at[wid, pl.ds(s * S, S)], src_v.at[buf], semis).wait()
            pltpu.make_async_copy(
                dst_hbm.at[wid, pl.ds(s * S, S)], dst_v.at[buf], semid).wait()

        def section(s, carry):
            sb = s % 2
            nb = (s + 1) % 2
            for jl in range(S):
                b = jl % K
                sem = gsems[b]
                pltpu.make_async_copy(
                    r_hbm.at[src_v.at[sb, jl]], rows_v.at[b], sem).wait()
                pltpu.sync_copy(rows_v.at[b], acc_sh.at[dst_v.at[sb, jl]],
                                add=True)
                if jl + K < S:
                    pltpu.async_copy(
                        r_hbm.at[src_v.at[sb, jl + K]], rows_v.at[b], sem)
                else:
                    if jl == S - K:
                        @pl.when(s + 1 < nsec)
                        def _():
                            idx_wait(s + 1, nb)

                    jn = jl + K - S

                    @pl.when(s + 1 < nsec)
                    def _():
                        pltpu.async_copy(
                            r_hbm.at[src_v.at[nb, jn]], rows_v.at[b], sem)
            @pl.when(s + 2 < nsec)
            def _():
                pltpu.async_copy(
                    src_hbm.at[wid, pl.ds((s + 2) * S, S)], src_v.at[sb],
                    semis)
                pltpu.async_copy(
                    dst_hbm.at[wid, pl.ds((s + 2) * S, S)], dst_v.at[sb],
                    semid)
            return carry

        with jax.named_scope("gin_mainloop"):
            lax.fori_loop(0, nsec, section, 0)
            plsc.subcore_barrier()

        with jax.named_scope("gin_wb"):
            pltpu.sync_copy(acc_sh.at[pl.ds(r0, rows_per_tile)],
                            out_hbm.at[cid, pl.ds(r0, rows_per_tile)])

    return sc_scatter


def kernel(x, edge_index, W1, b1, gamma, beta, W2, b2, eps):
    N, D = x.shape
    E = edge_index.shape[1]
    C = 80
    S = 8
    K = 4

    n_pad = ((N + 8 * N_TILES - 1) // (8 * N_TILES)) * (8 * N_TILES)

    grain = NW * C * S
    e_pad = ((E + grain - 1) // grain) * grain
    nchunk = e_pad // (NW * C)
    src_flat = jnp.concatenate(
        [edge_index[0], jnp.zeros((e_pad - E,), jnp.int32)])
    dst_flat = jnp.concatenate(
        [edge_index[1], jnp.full((e_pad - E,), N, jnp.int32)])
    src = src_flat.reshape(NW, nchunk, C)
    dst = dst_flat.reshape(NW, nchunk, C)
    zeros = jnp.zeros((n_pad // N_TILES, D), jnp.float32)

    r = pl.pallas_call(
        _relu_body,
        out_shape=jax.ShapeDtypeStruct((N, D), jnp.float32),
    )(x)

    partials = _make_sc_scatter(n_pad, D, nchunk, C, S, K)(r, src, dst,
                                                           zeros)

    out = pl.pallas_call(
        _mlp_body,
        out_shape=jax.ShapeDtypeStruct((N, D), jnp.float32),
    )(x, partials, W1.T, b1.reshape(1, D), gamma.reshape(1, D),
      beta.reshape(1, D), W2.T, b2.reshape(1, D), eps.reshape(1, 1))
    return out


# speedup vs baseline: 1.1232x; 1.1232x over previous
import functools

import jax
import jax.numpy as jnp
from jax import lax
from jax.experimental import pallas as pl
from jax.experimental.pallas import tpu as pltpu
from jax.experimental.pallas import tpu_sc as plsc

N_SC = 2
N_TILES = 16
NW = N_SC * N_TILES


def _relu_body(x_ref, o_ref):
    o_ref[...] = jnp.maximum(x_ref[...], 0.0)


def _mlp_body(x_ref, p_ref, w1t_ref, b1_ref, g_ref, bt_ref, w2t_ref,
              b2_ref, eps_ref, o_ref):
    n = x_ref.shape[0]
    h = (1.0 + eps_ref[...]) * x_ref[...] + p_ref[0, :n] + p_ref[1, :n]
    l1 = jnp.dot(h, w1t_ref[...], preferred_element_type=jnp.float32)
    l1 = l1 + b1_ref[...]
    mean = jnp.mean(l1, axis=0, keepdims=True)
    cen = l1 - mean
    var = jnp.mean(cen * cen, axis=0, keepdims=True)
    hn = cen * lax.rsqrt(var + 1e-5) * g_ref[...] + bt_ref[...]
    hn = jnp.maximum(hn, 0.0)
    o_ref[...] = (jnp.dot(hn, w2t_ref[...], preferred_element_type=jnp.float32)
                  + b2_ref[...])


def _make_sc_scatter(N, D, C, S, K, NC0, NC1):
    rows_per_tile = N // N_TILES
    assert S % K == 0 and NC0 % S == 0 and NC1 % S == 0
    mesh = plsc.VectorSubcoreMesh(
        core_axis_name="c", subcore_axis_name="s",
        num_cores=N_SC, num_subcores=N_TILES)

    @functools.partial(
        pl.kernel,
        out_type=jax.ShapeDtypeStruct((N_SC, N, D), jnp.float32),
        mesh=mesh,
        scratch_types=[
            pltpu.VMEM((2, S, C), jnp.int32),
            pltpu.VMEM((2, S, C), jnp.int32),
            pltpu.VMEM((K, C, D), jnp.float32),
            pltpu.VMEM_SHARED((N, D), jnp.float32),
            [pltpu.SemaphoreType.DMA] * K,
            pltpu.SemaphoreType.DMA,
            pltpu.SemaphoreType.DMA,
        ],
    )
    def sc_scatter(r_hbm, src_hbm, dst_hbm, z_hbm, out_hbm,
                   src_v, dst_v, rows_v, acc_sh, gsems, semis, semid):
        cid = lax.axis_index("c")
        sid = lax.axis_index("s")
        r0 = sid * rows_per_tile
        base = jnp.where(cid == 0, sid * NC0, N_TILES * NC0 + sid * NC1)
        nsec = jnp.where(cid == 0, NC0 // S, NC1 // S)

        with jax.named_scope("gin_zinit"):
            pltpu.sync_copy(z_hbm, acc_sh.at[pl.ds(r0, rows_per_tile)])

            pltpu.sync_copy(src_hbm.at[pl.ds(base, S)], src_v.at[0])
            pltpu.sync_copy(dst_hbm.at[pl.ds(base, S)], dst_v.at[0])
            pltpu.async_copy(src_hbm.at[pl.ds(base + S, S)], src_v.at[1],
                             semis)
            pltpu.async_copy(dst_hbm.at[pl.ds(base + S, S)], dst_v.at[1],
                             semid)
            plsc.subcore_barrier()

        for pb in range(K):
            pltpu.async_copy(r_hbm.at[src_v.at[0, pb]], rows_v.at[pb],
                             gsems[pb])

        def idx_wait(s, buf):
            pltpu.make_async_copy(
                src_hbm.at[pl.ds(base + s * S, S)], src_v.at[buf],
                semis).wait()
            pltpu.make_async_copy(
                dst_hbm.at[pl.ds(base + s * S, S)], dst_v.at[buf],
                semid).wait()

        def section(s, carry):
            sb = s % 2
            nb = (s + 1) % 2
            for jl in range(S):
                b = jl % K
                sem = gsems[b]
                pltpu.make_async_copy(
                    r_hbm.at[src_v.at[sb, jl]], rows_v.at[b], sem).wait()
                pltpu.sync_copy(rows_v.at[b], acc_sh.at[dst_v.at[sb, jl]],
                                add=True)
                if jl + K < S:
                    pltpu.async_copy(
                        r_hbm.at[src_v.at[sb, jl + K]], rows_v.at[b], sem)
                else:
                    if jl == S - K:
                        @pl.when(s + 1 < nsec)
                        def _():
                            idx_wait(s + 1, nb)

                    jn = jl + K - S

                    @pl.when(s + 1 < nsec)
                    def _():
                        pltpu.async_copy(
                            r_hbm.at[src_v.at[nb, jn]], rows_v.at[b], sem)
            @pl.when(s + 2 < nsec)
            def _():
                pltpu.async_copy(
                    src_hbm.at[pl.ds(base + (s + 2) * S, S)], src_v.at[sb],
                    semis)
                pltpu.async_copy(
                    dst_hbm.at[pl.ds(base + (s + 2) * S, S)], dst_v.at[sb],
                    semid)
            return carry

        with jax.named_scope("gin_mainloop"):
            lax.fori_loop(0, nsec, section, 0)
            plsc.subcore_barrier()

        with jax.named_scope("gin_wb"):
            pltpu.sync_copy(acc_sh.at[pl.ds(r0, rows_per_tile)],
                            out_hbm.at[cid, pl.ds(r0, rows_per_tile)])

    return sc_scatter


def kernel(x, edge_index, W1, b1, gamma, beta, W2, b2, eps):
    N, D = x.shape
    E = edge_index.shape[1]
    C = 80
    S = 8
    K = 4

    n_pad = ((N + 8 * N_TILES - 1) // (8 * N_TILES)) * (8 * N_TILES)

    grain = NW * C * S
    e_pad = ((E + grain - 1) // grain) * grain
    nchunk_total = e_pad // C
    per_tile = nchunk_total // N_TILES
    NC0 = (int(per_tile * 0.85) // S) * S
    NC1 = per_tile - NC0
    assert NC1 % S == 0 and NC1 >= 2 * S
    src_flat = jnp.concatenate(
        [edge_index[0], jnp.zeros((e_pad - E,), jnp.int32)])
    dst_flat = jnp.concatenate(
        [edge_index[1], jnp.full((e_pad - E,), N, jnp.int32)])
    src = src_flat.reshape(nchunk_total, C)
    dst = dst_flat.reshape(nchunk_total, C)
    zeros = jnp.zeros((n_pad // N_TILES, D), jnp.float32)

    r = pl.pallas_call(
        _relu_body,
        out_shape=jax.ShapeDtypeStruct((N, D), jnp.float32),
    )(x)

    partials = _make_sc_scatter(n_pad, D, C, S, K, NC0, NC1)(r, src, dst,
                                                             zeros)

    out = pl.pallas_call(
        _mlp_body,
        out_shape=jax.ShapeDtypeStruct((N, D), jnp.float32),
    )(x, partials, W1.T, b1.reshape(1, D), gamma.reshape(1, D),
      beta.reshape(1, D), W2.T, b2.reshape(1, D), eps.reshape(1, 1))
    return out


# speedup vs baseline: 2.5780x; 2.2952x over previous
import functools

import jax
import jax.numpy as jnp
from jax import lax
from jax.experimental import pallas as pl
from jax.experimental.pallas import tpu as pltpu
from jax.experimental.pallas import tpu_sc as plsc

N_SC = 2
N_TILES = 16


def _relu_body(x_ref, o_ref):
    d_half = x_ref.shape[1] // 2
    o_ref[0] = jnp.maximum(x_ref[:, :d_half], 0.0)
    o_ref[1] = jnp.maximum(x_ref[:, d_half:], 0.0)


def _mlp_body(x_ref, p_ref, w1t_ref, b1_ref, g_ref, bt_ref, w2t_ref,
              b2_ref, eps_ref, o_ref):
    n = x_ref.shape[0]
    agg = jnp.concatenate([p_ref[0, :n], p_ref[1, :n]], axis=-1)
    h = (1.0 + eps_ref[...]) * x_ref[...] + agg
    l1 = jnp.dot(h, w1t_ref[...], preferred_element_type=jnp.float32)
    l1 = l1 + b1_ref[...]
    mean = jnp.mean(l1, axis=0, keepdims=True)
    cen = l1 - mean
    var = jnp.mean(cen * cen, axis=0, keepdims=True)
    hn = cen * lax.rsqrt(var + 1e-5) * g_ref[...] + bt_ref[...]
    hn = jnp.maximum(hn, 0.0)
    o_ref[...] = (jnp.dot(hn, w2t_ref[...], preferred_element_type=jnp.float32)
                  + b2_ref[...])


def _make_sc_scatter(N, NP, DH, C, S, K, NCH):
    src_rows_per_tile = N // N_TILES
    acc_rows_per_tile = NP // N_TILES
    nsec = NCH // S
    assert S % K == 0 and NCH % S == 0 and nsec >= 2
    mesh = plsc.VectorSubcoreMesh(
        core_axis_name="c", subcore_axis_name="s",
        num_cores=N_SC, num_subcores=N_TILES)

    @functools.partial(
        pl.kernel,
        out_type=jax.ShapeDtypeStruct((N_SC, NP, DH), jnp.float32),
        mesh=mesh,
        compiler_params=pltpu.CompilerParams(use_tc_tiling_on_sc=False),
        scratch_types=[
            pltpu.VMEM((2, S, C), jnp.int32),
            pltpu.VMEM((2, S, C), jnp.int32),
            pltpu.VMEM((K, C, DH), jnp.float32),
            pltpu.VMEM_SHARED((N, DH), jnp.float32),
            pltpu.VMEM_SHARED((NP, DH), jnp.float32),
            [pltpu.SemaphoreType.DMA] * K,
            pltpu.SemaphoreType.DMA,
            pltpu.SemaphoreType.DMA,
        ],
    )
    def sc_scatter(rh_hbm, src_hbm, dst_hbm, z_hbm, out_hbm,
                   src_v, dst_v, rows_v, r_sh, acc_sh, gsems, semis, semid):
        cid = lax.axis_index("c")
        sid = lax.axis_index("s")
        base = sid * NCH
        a0 = sid * acc_rows_per_tile
        g0 = sid * src_rows_per_tile

        with jax.named_scope("gin_zinit"):
            pltpu.sync_copy(rh_hbm.at[cid, pl.ds(g0, src_rows_per_tile)],
                            r_sh.at[pl.ds(g0, src_rows_per_tile)])
            pltpu.sync_copy(z_hbm, acc_sh.at[pl.ds(a0, acc_rows_per_tile)])

            pltpu.sync_copy(src_hbm.at[pl.ds(base, S)], src_v.at[0])
            pltpu.sync_copy(dst_hbm.at[pl.ds(base, S)], dst_v.at[0])
            pltpu.async_copy(src_hbm.at[pl.ds(base + S, S)], src_v.at[1],
                             semis)
            pltpu.async_copy(dst_hbm.at[pl.ds(base + S, S)], dst_v.at[1],
                             semid)
            plsc.subcore_barrier()

        for pb in range(K):
            pltpu.async_copy(r_sh.at[src_v.at[0, pb]], rows_v.at[pb],
                             gsems[pb])

        def idx_wait(s, buf):
            pltpu.make_async_copy(
                src_hbm.at[pl.ds(base + s * S, S)], src_v.at[buf],
                semis).wait()
            pltpu.make_async_copy(
                dst_hbm.at[pl.ds(base + s * S, S)], dst_v.at[buf],
                semid).wait()

        def section(s, carry):
            sb = s % 2
            nb = (s + 1) % 2
            for jl in range(S):
                b = jl % K
                sem = gsems[b]
                pltpu.make_async_copy(
                    r_sh.at[src_v.at[sb, jl]], rows_v.at[b], sem).wait()
                pltpu.sync_copy(rows_v.at[b], acc_sh.at[dst_v.at[sb, jl]],
                                add=True)
                if jl + K < S:
                    pltpu.async_copy(
                        r_sh.at[src_v.at[sb, jl + K]], rows_v.at[b], sem)
                else:
                    if jl == S - K:
                        @pl.when(s + 1 < nsec)
                        def _():
                            idx_wait(s + 1, nb)

                    jn = jl + K - S

                    @pl.when(s + 1 < nsec)
                    def _():
                        pltpu.async_copy(
                            r_sh.at[src_v.at[nb, jn]], rows_v.at[b], sem)
            @pl.when(s + 2 < nsec)
            def _():
                pltpu.async_copy(
                    src_hbm.at[pl.ds(base + (s + 2) * S, S)], src_v.at[sb],
                    semis)
                pltpu.async_copy(
                    dst_hbm.at[pl.ds(base + (s + 2) * S, S)], dst_v.at[sb],
                    semid)
            return carry

        with jax.named_scope("gin_mainloop"):
            lax.fori_loop(0, nsec, section, 0)
            plsc.subcore_barrier()

        with jax.named_scope("gin_wb"):
            pltpu.sync_copy(acc_sh.at[pl.ds(a0, acc_rows_per_tile)],
                            out_hbm.at[cid, pl.ds(a0, acc_rows_per_tile)])

    return sc_scatter


def kernel(x, edge_index, W1, b1, gamma, beta, W2, b2, eps):
    N, D = x.shape
    E = edge_index.shape[1]
    DH = D // 2
    C = 128
    S = 8
    K = 4

    n_pad = ((N + 8 * N_TILES - 1) // (8 * N_TILES)) * (8 * N_TILES)

    grain = N_TILES * C * S
    e_pad = ((E + grain - 1) // grain) * grain
    nchunk_total = e_pad // C
    NCH = nchunk_total // N_TILES
    src_flat = jnp.concatenate(
        [edge_index[0], jnp.zeros((e_pad - E,), jnp.int32)])
    dst_flat = jnp.concatenate(
        [edge_index[1], jnp.full((e_pad - E,), N, jnp.int32)])
    src = src_flat.reshape(nchunk_total, C)
    dst = dst_flat.reshape(nchunk_total, C)
    zeros = jnp.zeros((n_pad // N_TILES, DH), jnp.float32)

    rh = pl.pallas_call(
        _relu_body,
        out_shape=jax.ShapeDtypeStruct((2, N, DH), jnp.float32),
    )(x)

    partials = _make_sc_scatter(N, n_pad, DH, C, S, K, NCH)(rh, src, dst,
                                                            zeros)

    out = pl.pallas_call(
        _mlp_body,
        out_shape=jax.ShapeDtypeStruct((N, D), jnp.float32),
    )(x, partials, W1.T, b1.reshape(1, D), gamma.reshape(1, D),
      beta.reshape(1, D), W2.T, b2.reshape(1, D), eps.reshape(1, 1))
    return out


# speedup vs baseline: 2.5923x; 1.0055x over previous
import functools

import jax
import jax.numpy as jnp
from jax import lax
from jax.experimental import pallas as pl
from jax.experimental.pallas import tpu as pltpu
from jax.experimental.pallas import tpu_sc as plsc

N_SC = 2
N_TILES = 16


def _relu_body(x_ref, o_ref):
    d_half = x_ref.shape[1] // 2
    o_ref[0] = jnp.maximum(x_ref[:, :d_half], 0.0)
    o_ref[1] = jnp.maximum(x_ref[:, d_half:], 0.0)


def _mlp_body(x_ref, p_ref, w1t_ref, b1_ref, g_ref, bt_ref, w2t_ref,
              b2_ref, eps_ref, o_ref):
    n = x_ref.shape[0]
    agg = jnp.concatenate([p_ref[0, :n], p_ref[1, :n]], axis=-1)
    h = (1.0 + eps_ref[...]) * x_ref[...] + agg
    l1 = jnp.dot(h, w1t_ref[...], preferred_element_type=jnp.float32)
    l1 = l1 + b1_ref[...]
    mean = jnp.mean(l1, axis=0, keepdims=True)
    cen = l1 - mean
    var = jnp.mean(cen * cen, axis=0, keepdims=True)
    hn = cen * lax.rsqrt(var + 1e-5) * g_ref[...] + bt_ref[...]
    hn = jnp.maximum(hn, 0.0)
    o_ref[...] = (jnp.dot(hn, w2t_ref[...], preferred_element_type=jnp.float32)
                  + b2_ref[...])


def _make_sc_scatter(N, NP, DH, C, S, K, NCH):
    src_rows_per_tile = N // N_TILES
    acc_rows_per_tile = NP // N_TILES
    nsec = NCH // S
    assert S % K == 0 and NCH % S == 0 and nsec >= 2
    mesh = plsc.VectorSubcoreMesh(
        core_axis_name="c", subcore_axis_name="s",
        num_cores=N_SC, num_subcores=N_TILES)

    @functools.partial(
        pl.kernel,
        out_type=jax.ShapeDtypeStruct((N_SC, NP, DH), jnp.float32),
        mesh=mesh,
        compiler_params=pltpu.CompilerParams(use_tc_tiling_on_sc=False),
        scratch_types=[
            pltpu.VMEM((2, S, C), jnp.int32),
            pltpu.VMEM((2, S, C), jnp.int32),
            pltpu.VMEM((K, C, DH), jnp.float32),
            pltpu.VMEM_SHARED((N, DH), jnp.float32),
            pltpu.VMEM_SHARED((NP, DH), jnp.float32),
            [pltpu.SemaphoreType.DMA] * K,
            [pltpu.SemaphoreType.DMA] * 2,
            pltpu.SemaphoreType.DMA,
            pltpu.SemaphoreType.DMA,
        ],
    )
    def sc_scatter(rh_hbm, src_hbm, dst_hbm, z_hbm, out_hbm,
                   src_v, dst_v, rows_v, r_sh, acc_sh, gsems, ssems,
                   semis, semid):
        cid = lax.axis_index("c")
        sid = lax.axis_index("s")
        base = sid * NCH
        a0 = sid * acc_rows_per_tile
        g0 = sid * src_rows_per_tile

        with jax.named_scope("gin_zinit"):
            pltpu.sync_copy(rh_hbm.at[cid, pl.ds(g0, src_rows_per_tile)],
                            r_sh.at[pl.ds(g0, src_rows_per_tile)])
            pltpu.sync_copy(z_hbm, acc_sh.at[pl.ds(a0, acc_rows_per_tile)])

            pltpu.sync_copy(src_hbm.at[pl.ds(base, S)], src_v.at[0])
            pltpu.sync_copy(dst_hbm.at[pl.ds(base, S)], dst_v.at[0])
            pltpu.async_copy(src_hbm.at[pl.ds(base + S, S)], src_v.at[1],
                             semis)
            pltpu.async_copy(dst_hbm.at[pl.ds(base + S, S)], dst_v.at[1],
                             semid)
            plsc.subcore_barrier()

        for pb in range(2):
            pltpu.async_copy(r_sh.at[src_v.at[0, pb]], rows_v.at[pb],
                             gsems[pb])

        def idx_wait(s, buf):
            pltpu.make_async_copy(
                src_hbm.at[pl.ds(base + s * S, S)], src_v.at[buf],
                semis).wait()
            pltpu.make_async_copy(
                dst_hbm.at[pl.ds(base + s * S, S)], dst_v.at[buf],
                semid).wait()

        def drain_scatter(sl):
            pltpu.make_async_copy(
                rows_v.at[sl], acc_sh.at[dst_v.at[0, 0]], ssems[sl]).wait()

        def section(s, carry):
            sb = s % 2
            nb = (s + 1) % 2
            for jl in range(S):
                b = jl % K
                sl = jl % 2
                pltpu.make_async_copy(
                    r_sh.at[src_v.at[sb, jl]], rows_v.at[b], gsems[b]).wait()
                if jl >= 1:
                    drain_scatter(1 - sl)
                else:
                    @pl.when(s > 0)
                    def _():
                        drain_scatter(1 - sl)
                pltpu.async_copy(rows_v.at[b], acc_sh.at[dst_v.at[sb, jl]],
                                 ssems[sl], add=True)
                gb = (jl + 2) % K
                if jl + 2 < S:
                    pltpu.async_copy(
                        r_sh.at[src_v.at[sb, jl + 2]], rows_v.at[gb],
                        gsems[gb])
                else:
                    if jl == S - 2:
                        @pl.when(s + 1 < nsec)
                        def _():
                            idx_wait(s + 1, nb)

                    jn = jl + 2 - S

                    @pl.when(s + 1 < nsec)
                    def _():
                        pltpu.async_copy(
                            r_sh.at[src_v.at[nb, jn]], rows_v.at[gb],
                            gsems[gb])
            @pl.when(s + 2 < nsec)
            def _():
                pltpu.async_copy(
                    src_hbm.at[pl.ds(base + (s + 2) * S, S)], src_v.at[sb],
                    semis)
                pltpu.async_copy(
                    dst_hbm.at[pl.ds(base + (s + 2) * S, S)], dst_v.at[sb],
                    semid)
            return carry

        with jax.named_scope("gin_mainloop"):
            lax.fori_loop(0, nsec, section, 0)
            drain_scatter((S - 1) % 2)
            plsc.subcore_barrier()

        with jax.named_scope("gin_wb"):
            pltpu.sync_copy(acc_sh.at[pl.ds(a0, acc_rows_per_tile)],
                            out_hbm.at[cid, pl.ds(a0, acc_rows_per_tile)])

    return sc_scatter


def kernel(x, edge_index, W1, b1, gamma, beta, W2, b2, eps):
    N, D = x.shape
    E = edge_index.shape[1]
    DH = D // 2
    C = 128
    S = 8
    K = 4

    n_pad = ((N + 8 * N_TILES - 1) // (8 * N_TILES)) * (8 * N_TILES)

    grain = N_TILES * C * S
    e_pad = ((E + grain - 1) // grain) * grain
    nchunk_total = e_pad // C
    NCH = nchunk_total // N_TILES
    src_flat = jnp.concatenate(
        [edge_index[0], jnp.zeros((e_pad - E,), jnp.int32)])
    dst_flat = jnp.concatenate(
        [edge_index[1], jnp.full((e_pad - E,), N, jnp.int32)])
    src = src_flat.reshape(nchunk_total, C)
    dst = dst_flat.reshape(nchunk_total, C)
    zeros = jnp.zeros((n_pad // N_TILES, DH), jnp.float32)

    rh = pl.pallas_call(
        _relu_body,
        out_shape=jax.ShapeDtypeStruct((2, N, DH), jnp.float32),
    )(x)

    partials = _make_sc_scatter(N, n_pad, DH, C, S, K, NCH)(rh, src, dst,
                                                            zeros)

    out = pl.pallas_call(
        _mlp_body,
        out_shape=jax.ShapeDtypeStruct((N, D), jnp.float32),
    )(x, partials, W1.T, b1.reshape(1, D), gamma.reshape(1, D),
      beta.reshape(1, D), W2.T, b2.reshape(1, D), eps.reshape(1, 1))
    return out
